# baseline (speedup 1.0000x reference)
import jax
import jax.numpy as jnp
from jax.experimental import pallas as pl

_H, _B, _N, _D, _C = 1, 64, 576, 256, 1024
_ROWS = _B * _N
_TILE = 4096
_GRID = _ROWS // _TILE


def _vq_kernel(x_ref, e_ref, dist_ref, ind_ref, quant_ref):
    x = x_ref[...]
    e = e_ref[...]
    dist = jax.lax.dot_general(
        x, e, (((1,), (1,)), ((), ())), preferred_element_type=jnp.float32)
    dist_ref[...] = dist
    ind = jnp.argmax(dist, axis=1).astype(jnp.int32)
    ind_ref[0, 0, :] = ind
    col = jax.lax.broadcasted_iota(jnp.int32, (_TILE, _C), 1)
    onehot = (col == ind[:, None]).astype(jnp.float32)
    quant_ref[...] = jax.lax.dot_general(
        onehot, e, (((1,), (0,)), ((), ())), preferred_element_type=jnp.float32)


def kernel(x, embed):
    x = x.astype(jnp.float32)
    xf = x.reshape(_ROWS, _D)
    e = embed.reshape(_C, _D)
    dist, ind3, quant = pl.pallas_call(
        _vq_kernel,
        grid=(_GRID,),
        in_specs=[
            pl.BlockSpec((_TILE, _D), lambda i: (i, 0)),
            pl.BlockSpec((_C, _D), lambda i: (0, 0)),
        ],
        out_specs=[
            pl.BlockSpec((_TILE, _C), lambda i: (i, 0)),
            pl.BlockSpec((1, 1, _TILE), lambda i: (i, 0, 0)),
            pl.BlockSpec((_TILE, _D), lambda i: (i, 0)),
        ],
        out_shape=[
            jax.ShapeDtypeStruct((_ROWS, _C), jnp.float32),
            jax.ShapeDtypeStruct((_GRID, 1, _TILE), jnp.int32),
            jax.ShapeDtypeStruct((_ROWS, _D), jnp.float32),
        ],
    )(xf, e)
    quantize = quant.reshape(_B, _N, _D)
    embed_ind = ind3.reshape(_B, _N)
    dist_out = dist.reshape(_H, _B, _N, _C)
    return (quantize, embed_ind, dist_out)
